# Initial kernel scaffold; baseline (speedup 1.0000x reference)
import functools

import jax
import jax.numpy as jnp
from jax import lax
from jax.experimental import pallas as pl
from jax.experimental.pallas import tpu as pltpu
from jax.experimental.pallas import tpu_sc as plsc

B = 4
N = 100000
E = 3200000
T = 2

NSC = 2
NTILES = 16
NSLICES = 8
E8 = E // NSLICES
C = 3200
R = C // 128
CH = E8 // C
NPAD = 100096
SL = NPAD // NTILES
SL_LAST = N - (NTILES - 1) * SL


def _tile_body(e_s, subs, objs, p2, out, table, sub_buf, obj_buf, p_buf,
               prod_buf, norm_buf, zero_buf, acc0, acc1):
    c = lax.axis_index("c")
    s = lax.axis_index("s")
    lb = s & 1
    b = 2 * c + lb
    slice_id = s >> 1

    def zinit(i, _):
        zero_buf[pl.ds(i * 16, 16)] = jnp.zeros((16,), jnp.float32)
        return 0
    lax.fori_loop(0, SL // 16, zinit, 0)
    off = s * SL
    pltpu.sync_copy(zero_buf, acc0.at[pl.ds(off, SL)])
    pltpu.sync_copy(zero_buf, acc1.at[pl.ds(off, SL)])
    plsc.subcore_barrier()

    for t in range(T):
        src = e_s if t == 0 else out
        pltpu.sync_copy(src.at[b], table)

        def chunk_body(g, _):
            base = slice_id * E8 + g * C
            rowbase = slice_id * (E8 // 128) + g * R
            pltpu.sync_copy(subs.at[pl.ds(base, C)], sub_buf)
            pltpu.sync_copy(objs.at[pl.ds(rowbase, R)], obj_buf)
            pltpu.sync_copy(p2.at[b * T + t, pl.ds(base, C)], p_buf)

            def inner(k, _):
                sub_v = sub_buf[pl.ds(k * 16, 16)]
                p_v = p_buf[pl.ds(k * 16, 16)]
                g_v = plsc.load_gather(table, [sub_v])
                prod_buf[pl.ds(k * 16, 16)] = g_v * p_v
                return 0
            lax.fori_loop(0, C // 16, inner, 0)

            def scat(j, _):
                vals = prod_buf.at[pl.ds(j * 128, 128)]
                idx = obj_buf.at[j]

                @pl.when(lb == 0)
                def _():
                    pltpu.sync_copy(vals, acc0.at[idx], add=True)

                @pl.when(lb == 1)
                def _():
                    pltpu.sync_copy(vals, acc1.at[idx], add=True)
                return 0
            lax.fori_loop(0, R, scat, 0)
            return 0
        lax.fori_loop(0, CH, chunk_body, 0)

        plsc.subcore_barrier()

        for lbn in (0, 1):
            acc = acc0 if lbn == 0 else acc1
            bn = 2 * c + lbn
            pltpu.sync_copy(acc.at[pl.ds(off, SL)], norm_buf)
            pltpu.sync_copy(zero_buf, acc.at[pl.ds(off, SL)])

            def nbody(i, _):
                x = norm_buf[pl.ds(i * 16, 16)]
                norm_buf[pl.ds(i * 16, 16)] = x / jnp.maximum(x, 1.0)
                return 0
            lax.fori_loop(0, SL // 16, nbody, 0)

            @pl.when(s < NTILES - 1)
            def _():
                pltpu.sync_copy(norm_buf, out.at[bn, pl.ds(off, SL)])

            @pl.when(s == NTILES - 1)
            def _():
                pltpu.sync_copy(norm_buf.at[pl.ds(0, SL_LAST)],
                                out.at[bn, pl.ds(off, SL_LAST)])
        plsc.subcore_barrier()


@jax.jit
def kernel(e_s, pair, p):
    subs = pair[:, 0]
    objs = pair[:, 1].reshape(E // 128, 128)
    p2 = p.reshape(B * T, E)

    mesh = plsc.VectorSubcoreMesh(core_axis_name="c", subcore_axis_name="s")
    f = pl.kernel(
        _tile_body,
        out_type=jax.ShapeDtypeStruct((B, N), jnp.float32),
        mesh=mesh,
        scratch_types=[
            pltpu.VMEM((N,), jnp.float32),
            pltpu.VMEM((C,), jnp.int32),
            pltpu.VMEM((R, 128), jnp.int32),
            pltpu.VMEM((C,), jnp.float32),
            pltpu.VMEM((C,), jnp.float32),
            pltpu.VMEM((SL,), jnp.float32),
            pltpu.VMEM((SL,), jnp.float32),
            pltpu.VMEM_SHARED((NPAD,), jnp.float32),
            pltpu.VMEM_SHARED((NPAD,), jnp.float32),
        ],
    )
    return f(e_s, subs, objs, p2)


# speedup vs baseline: 16.2905x; 16.2905x over previous
import jax
import jax.numpy as jnp
from jax import lax
from jax.experimental import pallas as pl
from jax.experimental.pallas import tpu as pltpu
from jax.experimental.pallas import tpu_sc as plsc

B = 4
N = 100000
E = 3200000
T = 2

NTILES = 16
NSLICES = 8
C = 1024
R = C // 128
NCHUNKS = E // C
NCH = -(-NCHUNKS // NSLICES)
NPAD = 100096
SL = NPAD // NTILES
SL_LAST = N - (NTILES - 1) * SL


def _tile_body(e_s, subs, objs, p3, out, table, sub_buf, obj_buf, p_buf,
               prod_buf, norm_buf, zero_buf, acc0, acc1):
    c = lax.axis_index("c")
    s = lax.axis_index("s")
    lb = s & 1
    b = 2 * c + lb
    slice_id = s >> 1

    def zinit(i, _):
        zero_buf[pl.ds(i * 16, 16)] = jnp.zeros((16,), jnp.float32)
        return 0
    lax.fori_loop(0, SL // 16, zinit, 0)
    off = s * SL
    pltpu.sync_copy(zero_buf, acc0.at[pl.ds(off, SL)])
    pltpu.sync_copy(zero_buf, acc1.at[pl.ds(off, SL)])
    plsc.subcore_barrier()

    for t in range(T):
        src = e_s if t == 0 else out
        pltpu.sync_copy(src.at[pl.ds(b * N, N)], table)

        def chunk_body(g, _):
            m = g * NSLICES + slice_id

            @pl.when(m < NCHUNKS)
            def _():
                base = m * C
                pltpu.sync_copy(subs.at[pl.ds(base, C)], sub_buf)
                pltpu.sync_copy(objs.at[pl.ds(m * R, R)], obj_buf)
                pltpu.sync_copy(p3.at[b, :, pl.ds(base, C)], p_buf)

                def inner(k, _):
                    sub_v = sub_buf[pl.ds(k * 16, 16)]
                    p_v = p_buf[t, pl.ds(k * 16, 16)]
                    g_v = plsc.load_gather(table, [sub_v])
                    prod_buf[pl.ds(k * 16, 16)] = g_v * p_v
                    return 0
                lax.fori_loop(0, C // 16, inner, 0)

                def scat(j, _):
                    vals = prod_buf.at[pl.ds(j * 128, 128)]
                    idx = obj_buf.at[j]

                    @pl.when(lb == 0)
                    def _():
                        pltpu.sync_copy(vals, acc0.at[idx], add=True)

                    @pl.when(lb == 1)
                    def _():
                        pltpu.sync_copy(vals, acc1.at[idx], add=True)
                    return 0
                lax.fori_loop(0, R, scat, 0)
            return 0
        lax.fori_loop(0, NCH, chunk_body, 0)

        plsc.subcore_barrier()

        for lbn in (0, 1):
            acc = acc0 if lbn == 0 else acc1
            bn = 2 * c + lbn
            pltpu.sync_copy(acc.at[pl.ds(off, SL)], norm_buf)
            pltpu.sync_copy(zero_buf, acc.at[pl.ds(off, SL)])

            def nbody(i, _):
                x = norm_buf[pl.ds(i * 16, 16)]
                norm_buf[pl.ds(i * 16, 16)] = x / jnp.maximum(x, 1.0)
                return 0
            lax.fori_loop(0, SL // 16, nbody, 0)

            @pl.when(s < NTILES - 1)
            def _():
                pltpu.sync_copy(norm_buf, out.at[pl.ds(bn * N + off, SL)])

            @pl.when(s == NTILES - 1)
            def _():
                pltpu.sync_copy(norm_buf.at[pl.ds(0, SL_LAST)],
                                out.at[pl.ds(bn * N + off, SL_LAST)])
        plsc.subcore_barrier()


@jax.jit
def kernel(e_s, pair, p):
    subs = pair[:, 0]
    objs = pair[:, 1].reshape(E // 128, 128)
    e_s1 = e_s.reshape(B * N)

    mesh = plsc.VectorSubcoreMesh(core_axis_name="c", subcore_axis_name="s")
    f = pl.kernel(
        _tile_body,
        out_type=jax.ShapeDtypeStruct((B * N,), jnp.float32),
        mesh=mesh,
        compiler_params=pltpu.CompilerParams(needs_layout_passes=False),
        scratch_types=[
            pltpu.VMEM((N,), jnp.float32),
            pltpu.VMEM((C,), jnp.int32),
            pltpu.VMEM((R, 128), jnp.int32),
            pltpu.VMEM((T, C), jnp.float32),
            pltpu.VMEM((C,), jnp.float32),
            pltpu.VMEM((SL,), jnp.float32),
            pltpu.VMEM((SL,), jnp.float32),
            pltpu.VMEM_SHARED((NPAD,), jnp.float32),
            pltpu.VMEM_SHARED((NPAD,), jnp.float32),
        ],
    )
    return f(e_s1, subs, objs, p).reshape(B, N)
